# baseline (speedup 1.0000x reference)
import functools

import jax
import jax.numpy as jnp
from jax import lax
from jax.experimental import pallas as pl
from jax.experimental.pallas import tpu as pltpu
from jax.experimental.pallas import tpu_sc as plsc

NC = 2
NS = 16
NW = NC * NS
LANES = 64
CH = 128

_MESH = dict(core_axis_name="c", subcore_axis_name="s", num_cores=NC,
             num_subcores=NS)


def _zero_rows(ref):
    nrows, ncols = ref.shape
    cpl = ncols // 16

    def body(i, _):
        r = i // cpl
        c = (i % cpl) * 16
        ref[r, pl.ds(c, 16)] = jnp.zeros((16,), jnp.float32)
        return 0
    lax.fori_loop(0, nrows * cpl, body, 0)


@functools.cache
def _sc_deg(np_, steps):
    seg = np_ // NS

    def body(dst3_hbm, degp_hbm, dst_v, ones_v, tmp_v, acc_sh):
        cid = lax.axis_index("c")
        sid = lax.axis_index("s")
        wid = cid * NS + sid
        for t in range(LANES // 16):
            ones_v[pl.ds(t * 16, 16)] = jnp.ones((16,), jnp.float32)

        def zb(i, _):
            tmp_v[pl.ds(i * 16, 16)] = jnp.zeros((16,), jnp.float32)
            return 0
        lax.fori_loop(0, seg // 16, zb, 0)
        pltpu.sync_copy(tmp_v, acc_sh.at[pl.ds(sid * seg, seg)])
        pltpu.sync_copy(dst3_hbm.at[wid], dst_v)
        plsc.subcore_barrier()

        def sb(j, _):
            pltpu.sync_copy(ones_v, acc_sh.at[dst_v.at[j]], add=True)
            return 0
        lax.fori_loop(0, steps, sb, 0)
        plsc.subcore_barrier()
        pltpu.sync_copy(acc_sh.at[pl.ds(sid * seg, seg)], tmp_v)
        pltpu.sync_copy(tmp_v, degp_hbm.at[cid, pl.ds(sid * seg, seg)])

    return pl.kernel(
        body,
        out_type=jax.ShapeDtypeStruct((NC, np_), jnp.float32),
        mesh=plsc.VectorSubcoreMesh(**_MESH),
        scratch_types=[
            pltpu.VMEM((steps, LANES), jnp.int32),
            pltpu.VMEM((LANES,), jnp.float32),
            pltpu.VMEM((seg,), jnp.float32),
            pltpu.VMEM_SHARED((np_,), jnp.float32),
        ],
    )


CB = 20


@functools.cache
def _sc_edge(n, np_, ss, d):
    rseg = np_ // NS

    def body(y_hbm, src4_hbm, dst4_hbm, accp_hbm,
             src_v, dst_v, rb0, rb1, rb2, rb3, acc_sh,
             semg0, semg1, semg2, semg3, semsc):
        cid = lax.axis_index("c")
        sid = lax.axis_index("s")
        wid = cid * NS + sid
        base = sid * rseg
        rb = (rb0, rb1, rb2, rb3)
        sems = (semg0, semg1, semg2, semg3)
        _zero_rows(rb0)
        nz = rseg // LANES
        for k in range(nz):
            pltpu.async_copy(rb0, acc_sh.at[pl.ds(base + k * LANES, LANES)],
                             semsc)
        for k in range(nz):
            pltpu.make_async_copy(
                rb0, acc_sh.at[pl.ds(base + k * LANES, LANES)], semsc).wait()
        plsc.subcore_barrier()

        def gather(j, buf):
            pltpu.async_copy(y_hbm.at[src_v.at[j]], buf, semg0)

        def superchunk(s, _):
            pltpu.sync_copy(src4_hbm.at[wid, s], src_v)
            pltpu.sync_copy(dst4_hbm.at[wid, s], dst_v)
            gather(0, rb[0])
            gather(1, rb[1])
            gather(2, rb[2])
            for j in range(CB):
                buf = rb[j % 4]
                pltpu.make_async_copy(y_hbm.at[src_v.at[j]], buf,
                                      semg0).wait()
                pltpu.sync_copy(buf, acc_sh.at[dst_v.at[j]], add=True)
                if j + 3 < CB:
                    gather(j + 3, rb[(j + 3) % 4])
            return 0
        lax.fori_loop(0, ss, superchunk, 0)
        plsc.subcore_barrier()

        def rd(k, buf):
            pltpu.async_copy(acc_sh.at[pl.ds(base + k * LANES, LANES)], buf,
                             sems[k % 4])

        def rd_wait(k, buf):
            pltpu.make_async_copy(
                acc_sh.at[pl.ds(base + k * LANES, LANES)], buf,
                sems[k % 4]).wait()

        def wr(k, buf):
            pltpu.async_copy(buf, accp_hbm.at[cid, pl.ds(base + k * LANES,
                                                         LANES)], semsc)

        def wr_wait(k, buf):
            pltpu.make_async_copy(
                buf, accp_hbm.at[cid, pl.ds(base + k * LANES, LANES)],
                semsc).wait()

        nr = rseg // LANES
        rd(0, rb[0])
        rd(1, rb[1])
        for k in range(nr):
            rd_wait(k, rb[k % 4])
            wr(k, rb[k % 4])
            if k + 2 < nr:
                if k >= 2:
                    wr_wait(k - 2, rb[(k - 2) % 4])
                rd(k + 2, rb[(k + 2) % 4])
        for k in range(max(0, nr - 4), nr):
            wr_wait(k, rb[k % 4])

    rows_t = pltpu.VMEM((LANES, d), jnp.float32)
    return pl.kernel(
        body,
        out_type=jax.ShapeDtypeStruct((NC, np_, d), jnp.float32),
        mesh=plsc.VectorSubcoreMesh(**_MESH),
        scratch_types=[
            pltpu.VMEM((CB, LANES), jnp.int32),
            pltpu.VMEM((CB, LANES), jnp.int32),
            rows_t, rows_t, rows_t, rows_t,
            pltpu.VMEM_SHARED((np_, d), jnp.float32),
            pltpu.SemaphoreType.DMA,
            pltpu.SemaphoreType.DMA,
            pltpu.SemaphoreType.DMA,
            pltpu.SemaphoreType.DMA,
            pltpu.SemaphoreType.DMA,
        ],
    )


def _dis_of(dpt_ref):
    deg = dpt_ref[:, 0:1] + dpt_ref[:, 1:2] + 1.0
    return lax.rsqrt(deg)


def _tc_first_body(x_ref, w_ref, dpt_ref, y_ref):
    dis = _dis_of(dpt_ref)
    y_ref[...] = jnp.dot(x_ref[...], w_ref[...],
                         preferred_element_type=jnp.float32) * dis


def _relu_state(a0_ref, a1_ref, yp_ref, b_ref, dpt_ref, extra=None):
    dis = _dis_of(dpt_ref)
    s = dis * (a0_ref[0] + a1_ref[0] + yp_ref[...]) + b_ref[...]
    if extra is not None:
        s = s + extra
    return jnp.maximum(s, 0.0), dis


def _tc_mid_body(a0_ref, a1_ref, yp_ref, b_ref, w_ref, dpt_ref,
                 h_ref, y_ref):
    h, dis = _relu_state(a0_ref, a1_ref, yp_ref, b_ref, dpt_ref)
    h_ref[...] = h
    y_ref[...] = jnp.dot(h, w_ref[...],
                         preferred_element_type=jnp.float32) * dis


def _tc_mid_res_body(a0_ref, a1_ref, yp_ref, hp_ref, b_ref, w_ref, dpt_ref,
                     h_ref, y_ref):
    h, dis = _relu_state(a0_ref, a1_ref, yp_ref, b_ref, dpt_ref,
                         extra=hp_ref[...])
    h_ref[...] = h
    y_ref[...] = jnp.dot(h, w_ref[...],
                         preferred_element_type=jnp.float32) * dis


def _tc_final_body(a0_ref, a1_ref, yp_ref, hp_ref, b_ref, dpt_ref,
                   batch_ref, wp_ref, bp_ref, out_ref, pool_sc, cnt_sc):
    i = pl.program_id(0)
    nblk = pl.num_programs(0)
    g = pool_sc.shape[0]

    @pl.when(i == 0)
    def _():
        pool_sc[...] = jnp.zeros_like(pool_sc)
        cnt_sc[...] = jnp.zeros_like(cnt_sc)

    h, _ = _relu_state(a0_ref, a1_ref, yp_ref, b_ref, dpt_ref,
                       extra=hp_ref[...])
    gids = lax.broadcasted_iota(jnp.int32, (g, h.shape[0]), 0)
    oh = (batch_ref[0] == gids).astype(jnp.float32)
    pool_sc[...] += jnp.dot(oh, h, preferred_element_type=jnp.float32)
    cnt_sc[...] += jnp.dot(oh, jnp.ones(h.shape, jnp.float32),
                           preferred_element_type=jnp.float32)

    @pl.when(i == nblk - 1)
    def _():
        pooled = pool_sc[...] / jnp.maximum(cnt_sc[...], 1.0)
        out_ref[...] = jnp.dot(pooled, wp_ref[...],
                               preferred_element_type=jnp.float32) + bp_ref[...]


def kernel(x, edge_index, batch, W0, b0, W1, b1, W2, b2, Wp, bp):
    n, d = x.shape
    e = edge_index.shape[1]
    g = 16
    nv = Wp.shape[1]
    blk = 1000
    nblk = n // blk

    epw = (e + NW - 1) // NW
    steps = (epw + LANES - 1) // LANES
    steps = ((steps + CB - 1) // CB) * CB
    ss = steps // CB
    ep = NW * steps * LANES
    np_ = ((n // NS) // CH + 1) * CH * NS
    src = edge_index[0].astype(jnp.int32)
    dst = edge_index[1].astype(jnp.int32)
    pad = ep - e
    src_p = jnp.concatenate([src, jnp.zeros((pad,), jnp.int32)])
    junk = n + jnp.arange(pad, dtype=jnp.int32) % (np_ - n)
    dst_p = jnp.concatenate([dst, junk])
    src4 = src_p.reshape(NW, ss, CB, LANES)
    dst4 = dst_p.reshape(NW, ss, CB, LANES)
    dst3 = dst_p.reshape(NW, steps, LANES)

    degp = _sc_deg(np_, steps)(dst3)
    dpt = degp[:, :n].T

    row = lambda i: (i, 0)
    full = lambda i: (0, 0)
    spec_nd = pl.BlockSpec((blk, d), row)
    spec_w = pl.BlockSpec((d, d), full)
    spec_dpt = pl.BlockSpec((blk, 2), row)
    spec_b = pl.BlockSpec((1, d), full)
    spec_a0 = pl.BlockSpec((1, blk, d), lambda i: (0, i, 0))
    spec_a1 = pl.BlockSpec((1, blk, d), lambda i: (1, i, 0))
    sds_nd = jax.ShapeDtypeStruct((n, d), jnp.float32)

    y0 = pl.pallas_call(
        _tc_first_body, grid=(nblk,),
        in_specs=[spec_nd, spec_w, spec_dpt],
        out_specs=spec_nd, out_shape=sds_nd,
    )(x, W0, dpt)

    edge_fn = _sc_edge(n, np_, ss, d)

    acc = edge_fn(y0, src4, dst4)
    h1, y1 = pl.pallas_call(
        _tc_mid_body, grid=(nblk,),
        in_specs=[spec_a0, spec_a1, spec_nd, spec_b, spec_w, spec_dpt],
        out_specs=(spec_nd, spec_nd), out_shape=(sds_nd, sds_nd),
    )(acc, acc, y0, b0.reshape(1, d), W1, dpt)

    acc = edge_fn(y1, src4, dst4)
    h2, y2 = pl.pallas_call(
        _tc_mid_res_body, grid=(nblk,),
        in_specs=[spec_a0, spec_a1, spec_nd, spec_nd, spec_b, spec_w,
                  spec_dpt],
        out_specs=(spec_nd, spec_nd), out_shape=(sds_nd, sds_nd),
    )(acc, acc, y1, h1, b1.reshape(1, d), W2, dpt)

    acc = edge_fn(y2, src4, dst4)
    batch3 = batch.astype(jnp.int32).reshape(nblk, 1, blk)
    out = pl.pallas_call(
        _tc_final_body, grid=(nblk,),
        in_specs=[spec_a0, spec_a1, spec_nd, spec_nd, spec_b, spec_dpt,
                  pl.BlockSpec((1, 1, blk), lambda i: (i, 0, 0)),
                  pl.BlockSpec((d, nv), full),
                  pl.BlockSpec((1, nv), full)],
        out_specs=pl.BlockSpec((g, nv), full),
        out_shape=jax.ShapeDtypeStruct((g, nv), jnp.float32),
        scratch_shapes=[pltpu.VMEM((g, d), jnp.float32),
                        pltpu.VMEM((g, d), jnp.float32)],
    )(acc, acc, y2, h2, b2.reshape(1, d), dpt, batch3, Wp,
      bp.reshape(1, nv))
    return out


# speedup vs baseline: 1.0364x; 1.0364x over previous
import functools

import jax
import jax.numpy as jnp
from jax import lax
from jax.experimental import pallas as pl
from jax.experimental.pallas import tpu as pltpu
from jax.experimental.pallas import tpu_sc as plsc

NC = 2
NS = 16
NW = NC * NS
LANES = 64
CH = 128

_MESH = dict(core_axis_name="c", subcore_axis_name="s", num_cores=NC,
             num_subcores=NS)


def _zero_rows(ref):
    nrows, ncols = ref.shape
    cpl = ncols // 16

    def body(i, _):
        r = i // cpl
        c = (i % cpl) * 16
        ref[r, pl.ds(c, 16)] = jnp.zeros((16,), jnp.float32)
        return 0
    lax.fori_loop(0, nrows * cpl, body, 0)


@functools.cache
def _sc_deg(np_, steps):
    seg = np_ // NS

    def body(dst3_hbm, degp_hbm, dst_v, ones_v, tmp_v, acc_sh):
        cid = lax.axis_index("c")
        sid = lax.axis_index("s")
        wid = cid * NS + sid
        for t in range(LANES // 16):
            ones_v[pl.ds(t * 16, 16)] = jnp.ones((16,), jnp.float32)

        def zb(i, _):
            tmp_v[pl.ds(i * 16, 16)] = jnp.zeros((16,), jnp.float32)
            return 0
        lax.fori_loop(0, seg // 16, zb, 0)
        pltpu.sync_copy(tmp_v, acc_sh.at[pl.ds(sid * seg, seg)])
        pltpu.sync_copy(dst3_hbm.at[wid], dst_v)
        plsc.subcore_barrier()

        def sb(j, _):
            pltpu.sync_copy(ones_v, acc_sh.at[dst_v.at[j]], add=True)
            return 0
        lax.fori_loop(0, steps, sb, 0)
        plsc.subcore_barrier()
        pltpu.sync_copy(acc_sh.at[pl.ds(sid * seg, seg)], tmp_v)
        pltpu.sync_copy(tmp_v, degp_hbm.at[cid, pl.ds(sid * seg, seg)])

    return pl.kernel(
        body,
        out_type=jax.ShapeDtypeStruct((NC, np_), jnp.float32),
        mesh=plsc.VectorSubcoreMesh(**_MESH),
        scratch_types=[
            pltpu.VMEM((steps, LANES), jnp.int32),
            pltpu.VMEM((LANES,), jnp.float32),
            pltpu.VMEM((seg,), jnp.float32),
            pltpu.VMEM_SHARED((np_,), jnp.float32),
        ],
    )


CB = 16


@functools.cache
def _sc_edge(n, np_, ss, d):
    rseg = np_ // NS

    def body(y_hbm, src4_hbm, dst4_hbm, accp_hbm,
             src_v, dst_v, rb0, rb1, rb2, rb3, acc_sh,
             semg0, semg1, semg2, semg3, semsc):
        cid = lax.axis_index("c")
        sid = lax.axis_index("s")
        wid = cid * NS + sid
        base = sid * rseg
        rb = (rb0, rb1, rb2, rb3)
        sems = (semg0, semg1, semg2, semg3)
        _zero_rows(rb0)
        nz = rseg // LANES
        for k in range(nz):
            pltpu.async_copy(rb0, acc_sh.at[pl.ds(base + k * LANES, LANES)],
                             semsc)
        for k in range(nz):
            pltpu.make_async_copy(
                rb0, acc_sh.at[pl.ds(base + k * LANES, LANES)], semsc).wait()
        plsc.subcore_barrier()

        def gather(j, buf):
            pltpu.async_copy(y_hbm.at[src_v.at[j]], buf, semg0)

        def superchunk(s, _):
            pltpu.sync_copy(src4_hbm.at[wid, s], src_v)
            pltpu.sync_copy(dst4_hbm.at[wid, s], dst_v)
            gather(0, rb[0])
            gather(1, rb[1])
            gather(2, rb[2])
            for j in range(CB):
                buf = rb[j % 4]
                pltpu.make_async_copy(y_hbm.at[src_v.at[j]], buf,
                                      semg0).wait()
                pltpu.sync_copy(buf, acc_sh.at[dst_v.at[j]], add=True)
                if j + 3 < CB:
                    gather(j + 3, rb[(j + 3) % 4])
            return 0
        lax.fori_loop(0, ss, superchunk, 0)
        plsc.subcore_barrier()

        def rd(k, buf):
            pltpu.async_copy(acc_sh.at[pl.ds(base + k * LANES, LANES)], buf,
                             sems[k % 4])

        def rd_wait(k, buf):
            pltpu.make_async_copy(
                acc_sh.at[pl.ds(base + k * LANES, LANES)], buf,
                sems[k % 4]).wait()

        def wr(k, buf):
            pltpu.async_copy(buf, accp_hbm.at[cid, pl.ds(base + k * LANES,
                                                         LANES)], semsc)

        def wr_wait(k, buf):
            pltpu.make_async_copy(
                buf, accp_hbm.at[cid, pl.ds(base + k * LANES, LANES)],
                semsc).wait()

        nr = rseg // LANES
        rd(0, rb[0])
        rd(1, rb[1])
        for k in range(nr):
            rd_wait(k, rb[k % 4])
            wr(k, rb[k % 4])
            if k + 2 < nr:
                if k >= 2:
                    wr_wait(k - 2, rb[(k - 2) % 4])
                rd(k + 2, rb[(k + 2) % 4])
        for k in range(max(0, nr - 4), nr):
            wr_wait(k, rb[k % 4])

    rows_t = pltpu.VMEM((LANES, d), jnp.float32)
    return pl.kernel(
        body,
        out_type=jax.ShapeDtypeStruct((NC, np_, d), jnp.float32),
        mesh=plsc.VectorSubcoreMesh(**_MESH),
        scratch_types=[
            pltpu.VMEM((CB, LANES), jnp.int32),
            pltpu.VMEM((CB, LANES), jnp.int32),
            rows_t, rows_t, rows_t, rows_t,
            pltpu.VMEM_SHARED((np_, d), jnp.float32),
            pltpu.SemaphoreType.DMA,
            pltpu.SemaphoreType.DMA,
            pltpu.SemaphoreType.DMA,
            pltpu.SemaphoreType.DMA,
            pltpu.SemaphoreType.DMA,
        ],
    )


def _dis_of(dpt_ref):
    deg = dpt_ref[:, 0:1] + dpt_ref[:, 1:2] + 1.0
    return lax.rsqrt(deg)


def _tc_first_body(x_ref, w_ref, dpt_ref, y_ref):
    dis = _dis_of(dpt_ref)
    y_ref[...] = jnp.dot(x_ref[...], w_ref[...],
                         preferred_element_type=jnp.float32) * dis


def _relu_state(a0_ref, a1_ref, yp_ref, b_ref, dpt_ref, extra=None):
    dis = _dis_of(dpt_ref)
    s = dis * (a0_ref[0] + a1_ref[0] + yp_ref[...]) + b_ref[...]
    if extra is not None:
        s = s + extra
    return jnp.maximum(s, 0.0), dis


def _tc_mid_body(a0_ref, a1_ref, yp_ref, b_ref, w_ref, dpt_ref,
                 h_ref, y_ref):
    h, dis = _relu_state(a0_ref, a1_ref, yp_ref, b_ref, dpt_ref)
    h_ref[...] = h
    y_ref[...] = jnp.dot(h, w_ref[...],
                         preferred_element_type=jnp.float32) * dis


def _tc_mid_res_body(a0_ref, a1_ref, yp_ref, hp_ref, b_ref, w_ref, dpt_ref,
                     h_ref, y_ref):
    h, dis = _relu_state(a0_ref, a1_ref, yp_ref, b_ref, dpt_ref,
                         extra=hp_ref[...])
    h_ref[...] = h
    y_ref[...] = jnp.dot(h, w_ref[...],
                         preferred_element_type=jnp.float32) * dis


def _tc_final_body(a0_ref, a1_ref, yp_ref, hp_ref, b_ref, dpt_ref,
                   batch_ref, wp_ref, bp_ref, out_ref, pool_sc, cnt_sc):
    i = pl.program_id(0)
    nblk = pl.num_programs(0)
    g = pool_sc.shape[0]

    @pl.when(i == 0)
    def _():
        pool_sc[...] = jnp.zeros_like(pool_sc)
        cnt_sc[...] = jnp.zeros_like(cnt_sc)

    h, _ = _relu_state(a0_ref, a1_ref, yp_ref, b_ref, dpt_ref,
                       extra=hp_ref[...])
    gids = lax.broadcasted_iota(jnp.int32, (g, h.shape[0]), 0)
    oh = (batch_ref[0] == gids).astype(jnp.float32)
    pool_sc[...] += jnp.dot(oh, h, preferred_element_type=jnp.float32)
    cnt_sc[...] += jnp.dot(oh, jnp.ones(h.shape, jnp.float32),
                           preferred_element_type=jnp.float32)

    @pl.when(i == nblk - 1)
    def _():
        pooled = pool_sc[...] / jnp.maximum(cnt_sc[...], 1.0)
        out_ref[...] = jnp.dot(pooled, wp_ref[...],
                               preferred_element_type=jnp.float32) + bp_ref[...]


def kernel(x, edge_index, batch, W0, b0, W1, b1, W2, b2, Wp, bp):
    n, d = x.shape
    e = edge_index.shape[1]
    g = 16
    nv = Wp.shape[1]
    blk = 1000
    nblk = n // blk

    epw = (e + NW - 1) // NW
    steps = (epw + LANES - 1) // LANES
    steps = ((steps + CB - 1) // CB) * CB
    ss = steps // CB
    ep = NW * steps * LANES
    np_ = ((n // NS) // CH + 1) * CH * NS
    src = edge_index[0].astype(jnp.int32)
    dst = edge_index[1].astype(jnp.int32)
    pad = ep - e
    src_p = jnp.concatenate([src, jnp.zeros((pad,), jnp.int32)])
    junk = n + jnp.arange(pad, dtype=jnp.int32) % (np_ - n)
    dst_p = jnp.concatenate([dst, junk])
    src4 = src_p.reshape(NW, ss, CB, LANES)
    dst4 = dst_p.reshape(NW, ss, CB, LANES)
    dst3 = dst_p.reshape(NW, steps, LANES)

    degp = _sc_deg(np_, steps)(dst3)
    dpt = degp[:, :n].T

    row = lambda i: (i, 0)
    full = lambda i: (0, 0)
    spec_nd = pl.BlockSpec((blk, d), row)
    spec_w = pl.BlockSpec((d, d), full)
    spec_dpt = pl.BlockSpec((blk, 2), row)
    spec_b = pl.BlockSpec((1, d), full)
    spec_a0 = pl.BlockSpec((1, blk, d), lambda i: (0, i, 0))
    spec_a1 = pl.BlockSpec((1, blk, d), lambda i: (1, i, 0))
    sds_nd = jax.ShapeDtypeStruct((n, d), jnp.float32)

    y0 = pl.pallas_call(
        _tc_first_body, grid=(nblk,),
        in_specs=[spec_nd, spec_w, spec_dpt],
        out_specs=spec_nd, out_shape=sds_nd,
    )(x, W0, dpt)

    edge_fn = _sc_edge(n, np_, ss, d)

    acc = edge_fn(y0, src4, dst4)
    h1, y1 = pl.pallas_call(
        _tc_mid_body, grid=(nblk,),
        in_specs=[spec_a0, spec_a1, spec_nd, spec_b, spec_w, spec_dpt],
        out_specs=(spec_nd, spec_nd), out_shape=(sds_nd, sds_nd),
    )(acc, acc, y0, b0.reshape(1, d), W1, dpt)

    acc = edge_fn(y1, src4, dst4)
    h2, y2 = pl.pallas_call(
        _tc_mid_res_body, grid=(nblk,),
        in_specs=[spec_a0, spec_a1, spec_nd, spec_nd, spec_b, spec_w,
                  spec_dpt],
        out_specs=(spec_nd, spec_nd), out_shape=(sds_nd, sds_nd),
    )(acc, acc, y1, h1, b1.reshape(1, d), W2, dpt)

    acc = edge_fn(y2, src4, dst4)
    batch3 = batch.astype(jnp.int32).reshape(nblk, 1, blk)
    out = pl.pallas_call(
        _tc_final_body, grid=(nblk,),
        in_specs=[spec_a0, spec_a1, spec_nd, spec_nd, spec_b, spec_dpt,
                  pl.BlockSpec((1, 1, blk), lambda i: (i, 0, 0)),
                  pl.BlockSpec((d, nv), full),
                  pl.BlockSpec((1, nv), full)],
        out_specs=pl.BlockSpec((g, nv), full),
        out_shape=jax.ShapeDtypeStruct((g, nv), jnp.float32),
        scratch_shapes=[pltpu.VMEM((g, d), jnp.float32),
                        pltpu.VMEM((g, d), jnp.float32)],
    )(acc, acc, y2, h2, b2.reshape(1, d), dpt, batch3, Wp,
      bp.reshape(1, nv))
    return out
